# Initial kernel scaffold; baseline (speedup 1.0000x reference)
import jax
import jax.numpy as jnp
from jax.experimental import pallas as pl


def kernel(x, edge_index, edge_attr, batch, ph_enc, temp_enc, box_idx, rec_flag, W_in, b_in, W_ep, b_ep, Wm1, bm1, Wm2, bm2, Wih, Whh, bih, bhh, ln_g, ln_b, Wcg, bcg, Wsg, bsg, box_table, Wcp1, bcp1, Wcp2, bcp2, Wmlp1, bmlp1, Wmlp2, bmlp2, Wmlp3, bmlp3):
    raise NotImplementedError("write your pallas kernel here")


# speedup vs baseline: 3.7743x; 3.7743x over previous
import functools

import jax
import jax.numpy as jnp
from jax import lax
from jax.experimental import pallas as pl
from jax.experimental.pallas import tpu as pltpu
from jax.experimental.pallas import tpu_sc as plsc

H = 128; EP = 16; NS = 3; NSTEPS = 4; COND = 11; NBOX = 8; BE = 8; MLPH = 256
N = 10000; E = 320000; D = 128; DE = 16; B = 64

NODE_BLK = 2000
EDGE_BLK = 2000
CHUNK = 128
NCHUNK = E // CHUNK
NWORK = 32
CPW = -(-NCHUNK // NWORK)
NPAD = 10240
SUB_ROWS = NPAD // 16


def _mt(a, b):
    return lax.dot_general(a, b, (((1,), (1,)), ((), ())),
                           preferred_element_type=jnp.float32)


def _mm(a, b):
    return lax.dot_general(a, b, (((1,), (0,)), ((), ())),
                           preferred_element_type=jnp.float32)


def _full(shape):
    return pl.BlockSpec(shape, lambda i: tuple(0 for _ in shape))


def _rows(blk, width):
    return pl.BlockSpec((blk, width), lambda i: (i, 0))


def _h0_body(x_ref, w_ref, b_ref, o_ref):
    o_ref[...] = _mt(x_ref[...], w_ref[...]) + b_ref[...]


def _h0_call(x, W_in, b_in2):
    return pl.pallas_call(
        _h0_body,
        grid=(N // NODE_BLK,),
        in_specs=[_rows(NODE_BLK, D), _full((H, D)), _full((1, H))],
        out_specs=_rows(NODE_BLK, H),
        out_shape=jax.ShapeDtypeStruct((N, H), jnp.float32),
    )(x, W_in, b_in2)


def _msg_body(ea_ref, wep_ref, bep_ref, w1_ref, b1_ref, w2_ref, b2_ref, o_ref):
    ef = _mt(ea_ref[...], wep_ref[...]) + bep_ref[...]
    t = jnp.maximum(_mt(ef, w1_ref[...]) + b1_ref[...], 0.0)
    o_ref[...] = _mt(t, w2_ref[...]) + b2_ref[...]


def _msg_call(edge_attr, W_ep, b_ep2, Wm1s, bm1s2, Wm2s, bm2s2):
    return pl.pallas_call(
        _msg_body,
        grid=(E // EDGE_BLK,),
        in_specs=[_rows(EDGE_BLK, DE), _full((EP, DE)), _full((1, EP)),
                  _full((H, EP)), _full((1, H)), _full((H, H)), _full((1, H))],
        out_specs=_rows(EDGE_BLK, H),
        out_shape=jax.ShapeDtypeStruct((E, H), jnp.float32),
    )(edge_attr, W_ep, b_ep2, Wm1s, bm1s2, Wm2s, bm2s2)


def _sc_agg(h, m, iidx, jidx):
    mesh = plsc.VectorSubcoreMesh(core_axis_name="c", subcore_axis_name="s")

    @functools.partial(
        pl.kernel,
        mesh=mesh,
        out_type=jax.ShapeDtypeStruct((2, NPAD, H), jnp.float32),
        scratch_types=[
            pltpu.VMEM((CHUNK,), jnp.int32),
            pltpu.VMEM((CHUNK,), jnp.int32),
            pltpu.VMEM((CHUNK, H), jnp.float32),
            pltpu.VMEM((CHUNK, H), jnp.float32),
            pltpu.VMEM_SHARED((NPAD, H), jnp.float32),
            pltpu.SemaphoreType.DMA,
            pltpu.SemaphoreType.DMA,
        ],
    )
    def k(h_hbm, m_hbm, i_hbm, j_hbm, out_hbm, jv, iv, gv, mv, acc, s1, s2):
        cid = lax.axis_index("c")
        sid = lax.axis_index("s")
        wid = sid * 2 + cid
        zero = jnp.zeros((16,), jnp.float32)

        def zrow(r, carry):
            for c8 in range(8):
                gv[r, pl.ds(c8 * 16, 16)] = zero
            return carry

        lax.fori_loop(0, CHUNK, zrow, 0)
        base = sid * SUB_ROWS
        for kk in range(SUB_ROWS // CHUNK):
            pltpu.sync_copy(gv, acc.at[pl.ds(base + kk * CHUNK, CHUNK)])
        plsc.subcore_barrier()

        def chunk(t, carry):
            c = wid + t * NWORK

            @pl.when(c < NCHUNK)
            def _():
                pltpu.sync_copy(j_hbm.at[pl.ds(c * CHUNK, CHUNK)], jv)
                pltpu.sync_copy(i_hbm.at[pl.ds(c * CHUNK, CHUNK)], iv)
                cp_m = pltpu.async_copy(m_hbm.at[pl.ds(c * CHUNK, CHUNK)], mv, s2)
                cp_g = pltpu.async_copy(h_hbm.at[jv], gv, s1)
                cp_g.wait()
                cp_m.wait()

                def mrow(r, cc):
                    for c8 in range(8):
                        sl = pl.ds(c8 * 16, 16)
                        gv[r, sl] = gv[r, sl] * mv[r, sl]
                    return cc

                lax.fori_loop(0, CHUNK, mrow, 0)
                pltpu.sync_copy(gv, acc.at[iv], add=True)

            return carry

        lax.fori_loop(0, CPW, chunk, 0)
        plsc.subcore_barrier()
        pltpu.sync_copy(acc.at[pl.ds(base, SUB_ROWS)],
                        out_hbm.at[cid, pl.ds(base, SUB_ROWS)])

    return k(h, m, iidx, jidx)


def _cond_feat(ph, te, rc, bx, btab):
    oh = (lax.broadcasted_iota(jnp.int32, (B, NBOX), 1) == bx).astype(jnp.float32)
    box_emb = _mm(oh, btab)
    return jnp.concatenate([ph, te, rc, box_emb], axis=1)


def _gru_body(ln, gate, *refs):
    if gate:
        (a0, a1, hr_, wih, whh, bih, bhh, lng, lnb, batchb,
         ph, te, rc, bx, btab, wcg, bcg, o) = refs
    elif ln:
        a0, a1, hr_, wih, whh, bih, bhh, lng, lnb, o = refs
    else:
        a0, a1, hr_, wih, whh, bih, bhh, o = refs
    h = hr_[...]
    agg = a0[0] + a1[0]
    gi = _mt(agg, wih[...]) + bih[...]
    gh = _mt(h, whh[...]) + bhh[...]
    r = jax.nn.sigmoid(gi[:, :H] + gh[:, :H])
    z = jax.nn.sigmoid(gi[:, H:2 * H] + gh[:, H:2 * H])
    nc = jnp.tanh(gi[:, 2 * H:] + r * gh[:, 2 * H:])
    hn = (1.0 - z) * nc + z * h
    if ln:
        mu = jnp.mean(hn, axis=1, keepdims=True)
        var = jnp.mean((hn - mu) ** 2, axis=1, keepdims=True)
        hn = (hn - mu) * lax.rsqrt(var + 1e-5) * lng[...] + lnb[...]
    if gate:
        cond = _cond_feat(ph[...], te[...], rc[...], bx[...], btab[...])
        g = jax.nn.sigmoid(_mt(cond, wcg[...]) + bcg[...])
        ohb = (lax.broadcasted_iota(jnp.int32, (hn.shape[0], B), 1)
               == batchb[...]).astype(jnp.float32)
        hn = hn * _mm(ohb, g)
    o[...] = hn


def _gru_call(ln, gate, parts, h, wih, whh, bih2, bhh2, lng2=None, lnb2=None,
              batch2=None, ph2=None, te2=None, rc2=None, bx2=None, btab=None,
              wcg=None, bcg2=None):
    args = [parts, parts, h, wih, whh, bih2, bhh2]
    specs = [pl.BlockSpec((1, NODE_BLK, H), lambda i: (0, i, 0)),
             pl.BlockSpec((1, NODE_BLK, H), lambda i: (1, i, 0)),
             _rows(NODE_BLK, H),
             _full((3 * H, H)), _full((3 * H, H)), _full((1, 3 * H)),
             _full((1, 3 * H))]
    if ln:
        args += [lng2, lnb2]
        specs += [_full((1, H)), _full((1, H))]
    if gate:
        args += [batch2, ph2, te2, rc2, bx2, btab, wcg, bcg2]
        specs += [_rows(NODE_BLK, 1), _full((B, 1)), _full((B, 1)),
                  _full((B, 1)), _full((B, 1)), _full((NBOX, BE)),
                  _full((H, COND)), _full((1, H))]
    return pl.pallas_call(
        functools.partial(_gru_body, ln, gate),
        grid=(N // NODE_BLK,),
        in_specs=specs,
        out_specs=_rows(NODE_BLK, H),
        out_shape=jax.ShapeDtypeStruct((N, H), jnp.float32),
    )(*args)


def _comb_body(s0, s1, s2, batchb, wsgp, bsgp, sums, cnts):
    cat = jnp.concatenate([s0[...], s1[...], s2[...]], axis=1)
    lg = _mt(cat, wsgp[...]) + bsgp[...]
    lanes = lax.broadcasted_iota(jnp.int32, lg.shape, 1)
    lg = jnp.where(lanes < NS, lg, -1e30)
    mx = jnp.max(lg, axis=1, keepdims=True)
    ex = jnp.exp(lg - mx)
    w = ex / jnp.sum(ex, axis=1, keepdims=True)
    hc = (w[:, 0:1] * s0[...] + w[:, 1:2] * s1[...] + w[:, 2:3] * s2[...])
    oh = (lax.broadcasted_iota(jnp.int32, (hc.shape[0], B), 1)
          == batchb[...]).astype(jnp.float32)
    ps = lax.dot_general(oh, hc, (((0,), (0,)), ((), ())),
                         preferred_element_type=jnp.float32)
    pc = lax.dot_general(oh, jnp.ones_like(hc), (((0,), (0,)), ((), ())),
                         preferred_element_type=jnp.float32)

    @pl.when(pl.program_id(0) == 0)
    def _():
        sums[...] = jnp.zeros_like(sums)
        cnts[...] = jnp.zeros_like(cnts)

    sums[...] += ps
    cnts[...] += pc


def _comb_call(s0, s1, s2, batch2, wsgp, bsgp2):
    return pl.pallas_call(
        _comb_body,
        grid=(N // NODE_BLK,),
        in_specs=[_rows(NODE_BLK, H), _rows(NODE_BLK, H), _rows(NODE_BLK, H),
                  _rows(NODE_BLK, 1), _full((8, NS * H)), _full((1, 8))],
        out_specs=[_full((B, H)), _full((B, H))],
        out_shape=[jax.ShapeDtypeStruct((B, H), jnp.float32),
                   jax.ShapeDtypeStruct((B, H), jnp.float32)],
    )(s0, s1, s2, batch2, wsgp, bsgp2)


def _head_body(sums, cnts, ph, te, rc, bx, btab, wcp1, bcp1, wcp2, bcp2,
               w1, b1, w2, b2, w3, b3, o):
    h_graph = sums[...] / jnp.maximum(cnts[...], 1.0)
    cond = _cond_feat(ph[...], te[...], rc[...], bx[...], btab[...])
    hc = jnp.maximum(_mt(cond, wcp1[...]) + bcp1[...], 0.0)
    hc = _mt(hc, wcp2[...]) + bcp2[...]
    cat = jnp.concatenate([h_graph, hc], axis=1)
    t = jnp.maximum(_mt(cat, w1[...]) + b1[...], 0.0)
    t = jnp.maximum(_mt(t, w2[...]) + b2[...], 0.0)
    o[...] = _mt(t, w3[...]) + b3[...]


def _head_call(sums, cnts, ph2, te2, rc2, bx2, btab, Wcp1, bcp12, Wcp2, bcp22,
               Wmlp1, bmlp12, Wmlp2, bmlp22, Wmlp3, bmlp32):
    return pl.pallas_call(
        _head_body,
        grid=(1,),
        in_specs=[_full((B, H)), _full((B, H)), _full((B, 1)), _full((B, 1)),
                  _full((B, 1)), _full((B, 1)), _full((NBOX, BE)),
                  _full((64, COND)), _full((1, 64)), _full((32, 64)),
                  _full((1, 32)), _full((MLPH, H + 32)), _full((1, MLPH)),
                  _full((H, MLPH)), _full((1, H)), _full((8, H)),
                  _full((1, 8))],
        out_specs=_full((B, 8)),
        out_shape=jax.ShapeDtypeStruct((B, 8), jnp.float32),
    )(sums, cnts, ph2, te2, rc2, bx2, btab, Wcp1, bcp12, Wcp2, bcp22,
      Wmlp1, bmlp12, Wmlp2, bmlp22, Wmlp3, bmlp32)


def kernel(x, edge_index, edge_attr, batch, ph_enc, temp_enc, box_idx,
           rec_flag, W_in, b_in, W_ep, b_ep, Wm1, bm1, Wm2, bm2, Wih, Whh,
           bih, bhh, ln_g, ln_b, Wcg, bcg, Wsg, bsg, box_table, Wcp1, bcp1,
           Wcp2, bcp2, Wmlp1, bmlp1, Wmlp2, bmlp2, Wmlp3, bmlp3):
    ei = edge_index.astype(jnp.int32)
    iidx = ei[0]
    jidx = ei[1]
    batch2 = batch.astype(jnp.int32).reshape(N, 1)
    ph2 = ph_enc.reshape(B, 1)
    te2 = temp_enc.reshape(B, 1)
    rc2 = rec_flag.reshape(B, 1)
    bx2 = box_idx.astype(jnp.int32).reshape(B, 1)

    row = lambda v: v.reshape(1, -1)
    wsgp = jnp.zeros((8, NS * H), jnp.float32).at[:NS].set(Wsg)
    bsgp2 = jnp.zeros((1, 8), jnp.float32).at[0, :NS].set(bsg)

    h = _h0_call(x, W_in, row(b_in))
    states = []
    for s in range(NS):
        m = _msg_call(edge_attr, W_ep, row(b_ep), Wm1[s], row(bm1[s]),
                      Wm2[s], row(bm2[s]))
        for st in range(NSTEPS):
            parts = _sc_agg(h, m, iidx, jidx)
            last = st == NSTEPS - 1
            if last:
                h = _gru_call(True, s == NS - 1, parts, h, Wih[s], Whh[s],
                              row(bih[s]), row(bhh[s]), row(ln_g[s]),
                              row(ln_b[s]), batch2, ph2, te2, rc2, bx2,
                              box_table, Wcg, row(bcg))
            else:
                h = _gru_call(False, False, parts, h, Wih[s], Whh[s],
                              row(bih[s]), row(bhh[s]))
        states.append(h)

    sums, cnts = _comb_call(states[0], states[1], states[2], batch2, wsgp,
                            bsgp2)
    w3p = jnp.zeros((8, H), jnp.float32).at[:1].set(Wmlp3)
    b3p = jnp.zeros((1, 8), jnp.float32).at[0, :1].set(bmlp3)
    out = _head_call(sums, cnts, ph2, te2, rc2, bx2, box_table, Wcp1,
                     row(bcp1), Wcp2, row(bcp2), Wmlp1, row(bmlp1), Wmlp2,
                     row(bmlp2), w3p, b3p)
    return out[:, :1]
